# baseline (speedup 1.0000x reference)
import functools

import jax
import jax.numpy as jnp
from jax import lax
from jax.experimental import pallas as pl
from jax.experimental.pallas import tpu as pltpu
from jax.experimental.pallas import tpu_sc as plsc

_NUM_CLASSES = 100000
_FEAT_DIM = 64
_BATCH = 16384
_LANES = 16
_NC = 2
_NS = 16
_NW = _NC * _NS
_BPW = _BATCH // _NW
_NGC = 4
_GC = _BPW // _NGC
_CHUNKS = _FEAT_DIM // _LANES

_mesh = plsc.VectorSubcoreMesh(core_axis_name="c", subcore_axis_name="s")


@functools.partial(
    pl.kernel,
    mesh=_mesh,
    out_type=jax.ShapeDtypeStruct((_NW, _LANES), jnp.float32),
    scratch_types=[
        pltpu.VMEM((_NGC, _GC), jnp.int32),
        pltpu.VMEM((_BPW, _FEAT_DIM), jnp.float32),
        pltpu.VMEM((_BPW, _FEAT_DIM), jnp.float32),
        pltpu.VMEM((_LANES,), jnp.float32),
        pltpu.SemaphoreType.DMA,
    ],
    compiler_params=pltpu.CompilerParams(use_tc_tiling_on_sc=False),
)
def _center_loss_sc(label_hbm, feature_hbm, centers_hbm, out_hbm,
                    idx_v, rows_v, feat_v, part_v, sem):
    wid = lax.axis_index("s") * _NC + lax.axis_index("c")

    pltpu.sync_copy(label_hbm.at[wid], idx_v)
    gathers = [
        pltpu.async_copy(
            centers_hbm.at[idx_v.at[j]],
            rows_v.at[pl.ds(j * _GC, _GC)],
            sem,
        )
        for j in range(_NGC)
    ]
    pltpu.sync_copy(feature_hbm.at[wid], feat_v)

    accs = tuple(jnp.zeros((_LANES,), jnp.float32) for _ in range(_CHUNKS))
    for j in range(_NGC):
        gathers[j].wait()

        def body(i, a, base=j * _GC):
            row = base + i
            new = []
            for c in range(_CHUNKS):
                sl = pl.ds(c * _LANES, _LANES)
                d = feat_v[row, sl] - rows_v[row, sl]
                new.append(a[c] + d * d)
            return tuple(new)

        accs = lax.fori_loop(0, _GC, body, accs)

    total = accs[0] + accs[1] + accs[2] + accs[3]
    part_v[...] = total
    pltpu.sync_copy(part_v, out_hbm.at[wid])


def kernel(label, feature, centers):
    lab = label.astype(jnp.int32).reshape(_NW, _NGC, _GC)
    feat = feature.reshape(_NW, _BPW, _FEAT_DIM)
    partials = _center_loss_sc(lab, feat, centers)
    return jnp.sum(partials) * 0.5
